# baseline (speedup 1.0000x reference)
import functools

import jax
import jax.numpy as jnp
from jax import lax
from jax.experimental import pallas as pl
from jax.experimental.pallas import tpu as pltpu
from jax.experimental.pallas import tpu_sc as plsc

NC = 2
NS = 16
L = 16
NW = NC * NS

B = 16384
D = 128
BPW = B // NW
CHUNK = 128
NCHUNK = BPW // CHUNK
GROUPS = CHUNK // L
NBUF = 2


def _sc_body(uids, iids, uemb, iemb, ubias, ibias, out,
             uidx_v, iidx_v, ubrow_v, ibrow_v,
             urows, irows, ubw, ibw, res_v, sems):
    wid = lax.axis_index("s") * NC + lax.axis_index("c")
    base = wid * BPW
    for c in range(NCHUNK):
        pltpu.sync_copy(uids.at[pl.ds(base + c * CHUNK, CHUNK)], uidx_v.at[c])
        pltpu.sync_copy(iids.at[pl.ds(base + c * CHUNK, CHUNK)], iidx_v.at[c])
    for c in range(NCHUNK):
        for g in range(GROUPS):
            idu = uidx_v[c, pl.ds(g * L, L)]
            idi = iidx_v[c, pl.ds(g * L, L)]
            ubrow_v[c, pl.ds(g * L, L)] = lax.shift_right_logical(idu, 4)
            ibrow_v[c, pl.ds(g * L, L)] = lax.shift_right_logical(idi, 4)

    def fire(c):
        s = c % NBUF
        return [
            pltpu.async_copy(uemb.at[uidx_v.at[c]], urows.at[s], sems.at[s]),
            pltpu.async_copy(iemb.at[iidx_v.at[c]], irows.at[s], sems.at[s]),
            pltpu.async_copy(ubias.at[ubrow_v.at[c]], ubw.at[s], sems.at[s]),
            pltpu.async_copy(ibias.at[ibrow_v.at[c]], ibw.at[s], sems.at[s]),
        ]

    handles = {0: fire(0)}
    for c in range(NCHUNK):
        s = c % NBUF
        if c + 1 < NCHUNK:
            handles[c + 1] = fire(c + 1)
        for h in handles.pop(c):
            h.wait()
        ur = urows.at[s]
        ir = irows.at[s]
        ubr = ubw.at[s]
        ibr = ibw.at[s]

        def group(g, carry, ur=ur, ir=ir, ubr=ubr, ibr=ibr, c=c):
            row = g * L + lax.iota(jnp.int32, L)
            acc = jnp.zeros((L,), jnp.float32)
            for d in range(D):
                dcol = jnp.full((L,), d, jnp.int32)
                u = plsc.load_gather(ur, [row, dcol])
                v = plsc.load_gather(ir, [row, dcol])
                acc = acc + u * v
            ucol = jnp.bitwise_and(uidx_v[c, pl.ds(g * L, L)], 15)
            icol = jnp.bitwise_and(iidx_v[c, pl.ds(g * L, L)], 15)
            ub = plsc.load_gather(ubr, [row, ucol])
            ib = plsc.load_gather(ibr, [row, icol])
            x = acc + ub + ib
            y = 1.0 / (1.0 + jnp.exp(-x))
            res_v[pl.ds(c * CHUNK + g * L, L)] = y
            return carry

        lax.fori_loop(0, GROUPS, group, 0)
    pltpu.sync_copy(res_v, out.at[pl.ds(base, BPW)])


_sc_kernel = functools.partial(
    pl.kernel,
    out_type=jax.ShapeDtypeStruct((B,), jnp.float32),
    mesh=plsc.VectorSubcoreMesh(core_axis_name="c", subcore_axis_name="s"),
    compiler_params=pltpu.CompilerParams(
        needs_layout_passes=False, use_tc_tiling_on_sc=False),
    scratch_types=[
        pltpu.VMEM((NCHUNK, CHUNK), jnp.int32),
        pltpu.VMEM((NCHUNK, CHUNK), jnp.int32),
        pltpu.VMEM((NCHUNK, CHUNK), jnp.int32),
        pltpu.VMEM((NCHUNK, CHUNK), jnp.int32),
        pltpu.VMEM((NBUF, CHUNK, D), jnp.float32),
        pltpu.VMEM((NBUF, CHUNK, D), jnp.float32),
        pltpu.VMEM((NBUF, CHUNK, L), jnp.float32),
        pltpu.VMEM((NBUF, CHUNK, L), jnp.float32),
        pltpu.VMEM((BPW,), jnp.float32),
        pltpu.SemaphoreType.DMA((NBUF,)),
    ],
)(_sc_body)


def kernel(user_ids, item_ids, user_emb, item_emb, user_bias, item_bias):
    ub16 = user_bias.reshape(-1, L)
    ib16 = item_bias.reshape(-1, L)
    out = _sc_kernel(user_ids.astype(jnp.int32), item_ids.astype(jnp.int32),
                     user_emb, item_emb, ub16, ib16)
    return out.reshape(B, 1)


# speedup vs baseline: 2.4737x; 2.4737x over previous
import functools

import jax
import jax.numpy as jnp
from jax import lax
from jax.experimental import pallas as pl
from jax.experimental.pallas import tpu as pltpu
from jax.experimental.pallas import tpu_sc as plsc

NC = 2
NS = 16
L = 16
NW = NC * NS

B = 16384
D = 128
BPW = B // NW
CHUNK = 128
NCHUNK = BPW // CHUNK
GROUPS = CHUNK // L
NBUF = 2
PITCH = L + 1


def _sc_body(uids, iids, uemb, iemb, ubias, ibias, out,
             uidx_v, iidx_v, ubrow_v, ibrow_v,
             urows, irows, ubw, ibw, accmat, res_v, sems):
    wid = lax.axis_index("s") * NC + lax.axis_index("c")
    base = wid * BPW
    for c in range(NCHUNK):
        pltpu.sync_copy(uids.at[pl.ds(base + c * CHUNK, CHUNK)], uidx_v.at[c])
        pltpu.sync_copy(iids.at[pl.ds(base + c * CHUNK, CHUNK)], iidx_v.at[c])
    for c in range(NCHUNK):
        for g in range(GROUPS):
            idu = uidx_v[c, pl.ds(g * L, L)]
            idi = iidx_v[c, pl.ds(g * L, L)]
            ubrow_v[c, pl.ds(g * L, L)] = lax.shift_right_logical(idu, 4)
            ibrow_v[c, pl.ds(g * L, L)] = lax.shift_right_logical(idi, 4)

    def fire(c):
        s = c % NBUF
        return [
            pltpu.async_copy(uemb.at[uidx_v.at[c]], urows.at[s], sems.at[s]),
            pltpu.async_copy(iemb.at[iidx_v.at[c]], irows.at[s], sems.at[s]),
            pltpu.async_copy(ubias.at[ubrow_v.at[c]], ubw.at[s], sems.at[s]),
            pltpu.async_copy(ibias.at[ibrow_v.at[c]], ibw.at[s], sems.at[s]),
        ]

    handles = {0: fire(0)}
    for c in range(NCHUNK):
        s = c % NBUF
        if c + 1 < NCHUNK:
            handles[c + 1] = fire(c + 1)
        for h in handles.pop(c):
            h.wait()
        ur = urows.at[s]
        ir = irows.at[s]
        ubr = ubw.at[s]
        ibr = ibw.at[s]

        def group(g, carry, ur=ur, ir=ir, ubr=ubr, ibr=ibr, c=c):
            row0 = g * L
            for r in range(L):
                acc = ur[row0 + r, pl.ds(0, L)] * ir[row0 + r, pl.ds(0, L)]
                for j in range(1, D // L):
                    acc = acc + (ur[row0 + r, pl.ds(j * L, L)]
                                 * ir[row0 + r, pl.ds(j * L, L)])
                accmat[pl.ds(r * PITCH, L)] = acc
            lanes = lax.iota(jnp.int32, L) * PITCH
            x = plsc.load_gather(accmat, [lanes])
            for l in range(1, L):
                x = x + plsc.load_gather(accmat, [lanes + l])
            row = row0 + lax.iota(jnp.int32, L)
            ucol = jnp.bitwise_and(uidx_v[c, pl.ds(g * L, L)], 15)
            icol = jnp.bitwise_and(iidx_v[c, pl.ds(g * L, L)], 15)
            ub = plsc.load_gather(ubr, [row, ucol])
            ib = plsc.load_gather(ibr, [row, icol])
            x = x + ub + ib
            y = 1.0 / (1.0 + jnp.exp(-x))
            res_v[pl.ds(c * CHUNK + g * L, L)] = y
            return carry

        lax.fori_loop(0, GROUPS, group, 0)
    pltpu.sync_copy(res_v, out.at[pl.ds(base, BPW)])


_sc_kernel = functools.partial(
    pl.kernel,
    out_type=jax.ShapeDtypeStruct((B,), jnp.float32),
    mesh=plsc.VectorSubcoreMesh(core_axis_name="c", subcore_axis_name="s"),
    compiler_params=pltpu.CompilerParams(
        needs_layout_passes=False, use_tc_tiling_on_sc=False),
    scratch_types=[
        pltpu.VMEM((NCHUNK, CHUNK), jnp.int32),
        pltpu.VMEM((NCHUNK, CHUNK), jnp.int32),
        pltpu.VMEM((NCHUNK, CHUNK), jnp.int32),
        pltpu.VMEM((NCHUNK, CHUNK), jnp.int32),
        pltpu.VMEM((NBUF, CHUNK, D), jnp.float32),
        pltpu.VMEM((NBUF, CHUNK, D), jnp.float32),
        pltpu.VMEM((NBUF, CHUNK, L), jnp.float32),
        pltpu.VMEM((NBUF, CHUNK, L), jnp.float32),
        pltpu.VMEM((L * PITCH,), jnp.float32),
        pltpu.VMEM((BPW,), jnp.float32),
        pltpu.SemaphoreType.DMA((NBUF,)),
    ],
)(_sc_body)


def kernel(user_ids, item_ids, user_emb, item_emb, user_bias, item_bias):
    ub16 = user_bias.reshape(-1, L)
    ib16 = item_bias.reshape(-1, L)
    out = _sc_kernel(user_ids.astype(jnp.int32), item_ids.astype(jnp.int32),
                     user_emb, item_emb, ub16, ib16)
    return out.reshape(B, 1)


# speedup vs baseline: 2.4741x; 1.0001x over previous
import functools

import jax
import jax.numpy as jnp
from jax import lax
from jax.experimental import pallas as pl
from jax.experimental.pallas import tpu as pltpu
from jax.experimental.pallas import tpu_sc as plsc

NC = 2
NS = 16
L = 16
NW = NC * NS

B = 16384
D = 128
BPW = B // NW
CHUNK = 128
NCHUNK = BPW // CHUNK
GROUPS = CHUNK // L
NBUF = 2
PITCH = L + 1


def _sc_body(uids, iids, uemb, iemb, ubias, ibias, out,
             uidx_v, iidx_v, ubrow_v, ibrow_v,
             urows, irows, ubw, ibw, accmat, res_v, sems):
    wid = lax.axis_index("s") * NC + lax.axis_index("c")
    base = wid * BPW
    for c in range(NCHUNK):
        pltpu.sync_copy(uids.at[pl.ds(base + c * CHUNK, CHUNK)], uidx_v.at[c])
        pltpu.sync_copy(iids.at[pl.ds(base + c * CHUNK, CHUNK)], iidx_v.at[c])
    for c in range(NCHUNK):
        for g in range(GROUPS):
            idu = uidx_v[c, pl.ds(g * L, L)]
            idi = iidx_v[c, pl.ds(g * L, L)]
            ubrow_v[c, pl.ds(g * L, L)] = lax.shift_right_logical(idu, 4)
            ibrow_v[c, pl.ds(g * L, L)] = lax.shift_right_logical(idi, 4)

    def fire(c):
        s = c % NBUF
        return [
            pltpu.async_copy(uemb.at[uidx_v.at[c]], urows.at[s], sems.at[s]),
            pltpu.async_copy(iemb.at[iidx_v.at[c]], irows.at[s], sems.at[s]),
            pltpu.async_copy(ubias.at[ubrow_v.at[c]], ubw.at[s], sems.at[s]),
            pltpu.async_copy(ibias.at[ibrow_v.at[c]], ibw.at[s], sems.at[s]),
        ]

    handles = {0: fire(0)}
    for c in range(NCHUNK):
        s = c % NBUF
        if c + 1 < NCHUNK:
            handles[c + 1] = fire(c + 1)
        for h in handles.pop(c):
            h.wait()
        ur = urows.at[s]
        ir = irows.at[s]
        ubr = ubw.at[s]
        ibr = ibw.at[s]

        def group(g, carry, ur=ur, ir=ir, ubr=ubr, ibr=ibr, c=c):
            row0 = g * L
            for r in range(L):
                acc = ur[row0 + r, pl.ds(0, L)] * ir[row0 + r, pl.ds(0, L)]
                for j in range(1, D // L):
                    acc = acc + (ur[row0 + r, pl.ds(j * L, L)]
                                 * ir[row0 + r, pl.ds(j * L, L)])
                accmat[pl.ds(r * PITCH, L)] = acc
            lanes = lax.iota(jnp.int32, L) * PITCH
            x = plsc.load_gather(accmat, [lanes])
            for l in range(1, L):
                x = x + plsc.load_gather(accmat, [lanes + l])
            row = row0 + lax.iota(jnp.int32, L)
            ucol = jnp.bitwise_and(uidx_v[c, pl.ds(g * L, L)], 15)
            icol = jnp.bitwise_and(iidx_v[c, pl.ds(g * L, L)], 15)
            ub = plsc.load_gather(ubr, [row, ucol])
            ib = plsc.load_gather(ibr, [row, icol])
            x = x + ub + ib
            y = 1.0 / (1.0 + jnp.exp(-x))
            res_v[pl.ds(c * CHUNK + g * L, L)] = y
            return carry

        lax.fori_loop(0, GROUPS, group, 0)
    pltpu.sync_copy(res_v, out.at[pl.ds(base, BPW)])


_sc_kernel = functools.partial(
    pl.kernel,
    out_type=jax.ShapeDtypeStruct((B,), jnp.float32),
    mesh=plsc.VectorSubcoreMesh(core_axis_name="c", subcore_axis_name="s"),
    compiler_params=pltpu.CompilerParams(
        needs_layout_passes=False, use_tc_tiling_on_sc=False,
        skip_device_barrier=True, disable_bounds_checks=True,
        disable_semaphore_checks=True),
    scratch_types=[
        pltpu.VMEM((NCHUNK, CHUNK), jnp.int32),
        pltpu.VMEM((NCHUNK, CHUNK), jnp.int32),
        pltpu.VMEM((NCHUNK, CHUNK), jnp.int32),
        pltpu.VMEM((NCHUNK, CHUNK), jnp.int32),
        pltpu.VMEM((NBUF, CHUNK, D), jnp.float32),
        pltpu.VMEM((NBUF, CHUNK, D), jnp.float32),
        pltpu.VMEM((NBUF, CHUNK, L), jnp.float32),
        pltpu.VMEM((NBUF, CHUNK, L), jnp.float32),
        pltpu.VMEM((L * PITCH,), jnp.float32),
        pltpu.VMEM((BPW,), jnp.float32),
        pltpu.SemaphoreType.DMA((NBUF,)),
    ],
)(_sc_body)


def kernel(user_ids, item_ids, user_emb, item_emb, user_bias, item_bias):
    ub16 = user_bias.reshape(-1, L)
    ib16 = item_bias.reshape(-1, L)
    out = _sc_kernel(user_ids.astype(jnp.int32), item_ids.astype(jnp.int32),
                     user_emb, item_emb, ub16, ib16)
    return out.reshape(B, 1)


# speedup vs baseline: 2.4746x; 1.0002x over previous
import functools

import jax
import jax.numpy as jnp
from jax import lax
from jax.experimental import pallas as pl
from jax.experimental.pallas import tpu as pltpu
from jax.experimental.pallas import tpu_sc as plsc

NC = 2
NS = 16
L = 16
NW = NC * NS

B = 16384
D = 128
BPW = B // NW
CHUNK = 128
NCHUNK = BPW // CHUNK
GROUPS = CHUNK // L
NBUF = 2
PITCH = L + 1


def _sc_body(uids, iids, uemb, iemb, ubias, ibias, out,
             uidx_v, iidx_v, ubrow_v, ibrow_v,
             urows, irows, ubw, ibw, accmat, res_v, sems):
    wid = lax.axis_index("s") * NC + lax.axis_index("c")
    base = wid * BPW
    for c in range(NCHUNK):
        pltpu.sync_copy(uids.at[pl.ds(base + c * CHUNK, CHUNK)], uidx_v.at[c])
        pltpu.sync_copy(iids.at[pl.ds(base + c * CHUNK, CHUNK)], iidx_v.at[c])
    for c in range(NCHUNK):
        for g in range(GROUPS):
            idu = uidx_v[c, pl.ds(g * L, L)]
            idi = iidx_v[c, pl.ds(g * L, L)]
            ubrow_v[c, pl.ds(g * L, L)] = lax.shift_right_logical(idu, 4)
            ibrow_v[c, pl.ds(g * L, L)] = lax.shift_right_logical(idi, 4)

    def fire(c):
        s = c % NBUF
        return [
            pltpu.async_copy(uemb.at[uidx_v.at[c]], urows.at[s], sems.at[s]),
            pltpu.async_copy(iemb.at[iidx_v.at[c]], irows.at[s], sems.at[s]),
            pltpu.async_copy(ubias.at[ubrow_v.at[c]], ubw.at[s], sems.at[s]),
            pltpu.async_copy(ibias.at[ibrow_v.at[c]], ibw.at[s], sems.at[s]),
        ]

    handles = {0: fire(0)}
    for c in range(NCHUNK):
        s = c % NBUF
        if c + 1 < NCHUNK:
            handles[c + 1] = fire(c + 1)
        for h in handles.pop(c):
            h.wait()
        ur = urows.at[s]
        ir = irows.at[s]
        ubr = ubw.at[s]
        ibr = ibw.at[s]

        def group(g, carry, ur=ur, ir=ir, ubr=ubr, ibr=ibr, c=c):
            row0 = g * L
            for r in range(L):
                acc = ur[row0 + r, pl.ds(0, L)] * ir[row0 + r, pl.ds(0, L)]
                for j in range(1, D // L):
                    acc = acc + (ur[row0 + r, pl.ds(j * L, L)]
                                 * ir[row0 + r, pl.ds(j * L, L)])
                accmat[pl.ds(r * PITCH, L)] = acc
            lanes = lax.iota(jnp.int32, L) * PITCH
            x = plsc.load_gather(accmat, [lanes])
            for l in range(1, L):
                x = x + plsc.load_gather(accmat, [lanes + l])
            row = row0 + lax.iota(jnp.int32, L)
            zero = jnp.zeros((L,), jnp.int32)
            ucol = jnp.bitwise_and(uidx_v[c, pl.ds(g * L, L)], 15)
            icol = jnp.bitwise_and(iidx_v[c, pl.ds(g * L, L)], 15)
            ub = plsc.load_gather(ubr, [row, zero, ucol])
            ib = plsc.load_gather(ibr, [row, zero, icol])
            x = x + ub + ib
            y = 1.0 / (1.0 + jnp.exp(-x))
            res_v[pl.ds(c * CHUNK + g * L, L)] = y
            return carry

        lax.fori_loop(0, GROUPS, group, 0)
    pltpu.sync_copy(res_v, out.at[pl.ds(base, BPW)])


_sc_kernel = functools.partial(
    pl.kernel,
    out_type=jax.ShapeDtypeStruct((B,), jnp.float32),
    mesh=plsc.VectorSubcoreMesh(core_axis_name="c", subcore_axis_name="s"),
    compiler_params=pltpu.CompilerParams(
        needs_layout_passes=False, use_tc_tiling_on_sc=False,
        skip_device_barrier=True, disable_bounds_checks=True,
        disable_semaphore_checks=True),
    scratch_types=[
        pltpu.VMEM((NCHUNK, CHUNK), jnp.int32),
        pltpu.VMEM((NCHUNK, CHUNK), jnp.int32),
        pltpu.VMEM((NCHUNK, CHUNK), jnp.int32),
        pltpu.VMEM((NCHUNK, CHUNK), jnp.int32),
        pltpu.VMEM((NBUF, CHUNK, D), jnp.float32),
        pltpu.VMEM((NBUF, CHUNK, D), jnp.float32),
        pltpu.VMEM((NBUF, CHUNK, 1, L), jnp.float32),
        pltpu.VMEM((NBUF, CHUNK, 1, L), jnp.float32),
        pltpu.VMEM((L * PITCH,), jnp.float32),
        pltpu.VMEM((BPW,), jnp.float32),
        pltpu.SemaphoreType.DMA((NBUF,)),
    ],
)(_sc_body)


def kernel(user_ids, item_ids, user_emb, item_emb, user_bias, item_bias):
    ub16 = user_bias.reshape(-1, 1, L)
    ib16 = item_bias.reshape(-1, 1, L)
    out = _sc_kernel(user_ids.astype(jnp.int32), item_ids.astype(jnp.int32),
                     user_emb, item_emb, ub16, ib16)
    return out.reshape(B, 1)


# speedup vs baseline: 2.5995x; 1.0504x over previous
import functools

import jax
import jax.numpy as jnp
from jax import lax
from jax.experimental import pallas as pl
from jax.experimental.pallas import tpu as pltpu
from jax.experimental.pallas import tpu_sc as plsc

NC = 2
NS = 16
L = 16
NW = NC * NS

B = 16384
D = 128
BPW = B // NW
CHUNK = 128
NCHUNK = BPW // CHUNK
GROUPS = CHUNK // L
NBUF = 2
PITCH = L + 1


def _sc_body(uids, iids, uemb, iemb, ubias, ibias, out,
             uidx_v, iidx_v, ubrow_v, ibrow_v,
             urows, irows, ubw, ibw, accmat, res_v, sems):
    wid = lax.axis_index("s") * NC + lax.axis_index("c")
    base = wid * BPW
    def stage_ids(c):
        pltpu.sync_copy(uids.at[pl.ds(base + c * CHUNK, CHUNK)], uidx_v.at[c])
        pltpu.sync_copy(iids.at[pl.ds(base + c * CHUNK, CHUNK)], iidx_v.at[c])
        for g in range(GROUPS):
            idu = uidx_v[c, pl.ds(g * L, L)]
            idi = iidx_v[c, pl.ds(g * L, L)]
            ubrow_v[c, pl.ds(g * L, L)] = lax.shift_right_logical(idu, 4)
            ibrow_v[c, pl.ds(g * L, L)] = lax.shift_right_logical(idi, 4)

    def fire(c):
        s = c % NBUF
        return [
            pltpu.async_copy(uemb.at[uidx_v.at[c]], urows.at[s], sems.at[s]),
            pltpu.async_copy(iemb.at[iidx_v.at[c]], irows.at[s], sems.at[s]),
            pltpu.async_copy(ubias.at[ubrow_v.at[c]], ubw.at[s], sems.at[s]),
            pltpu.async_copy(ibias.at[ibrow_v.at[c]], ibw.at[s], sems.at[s]),
        ]

    stage_ids(0)
    handles = {0: fire(0)}
    for c in range(NCHUNK):
        s = c % NBUF
        if c + 1 < NCHUNK:
            stage_ids(c + 1)
            handles[c + 1] = fire(c + 1)
        for h in handles.pop(c):
            h.wait()
        ur = urows.at[s]
        ir = irows.at[s]
        ubr = ubw.at[s]
        ibr = ibw.at[s]

        def group(g, carry, ur=ur, ir=ir, ubr=ubr, ibr=ibr, c=c):
            row0 = g * L
            for r in range(L):
                acc = ur[row0 + r, pl.ds(0, L)] * ir[row0 + r, pl.ds(0, L)]
                for j in range(1, D // L):
                    acc = acc + (ur[row0 + r, pl.ds(j * L, L)]
                                 * ir[row0 + r, pl.ds(j * L, L)])
                accmat[pl.ds(r * PITCH, L)] = acc
            lanes = lax.iota(jnp.int32, L) * PITCH
            x = plsc.load_gather(accmat, [lanes])
            for l in range(1, L):
                x = x + plsc.load_gather(accmat, [lanes + l])
            row = row0 + lax.iota(jnp.int32, L)
            zero = jnp.zeros((L,), jnp.int32)
            ucol = jnp.bitwise_and(uidx_v[c, pl.ds(g * L, L)], 15)
            icol = jnp.bitwise_and(iidx_v[c, pl.ds(g * L, L)], 15)
            ub = plsc.load_gather(ubr, [row, zero, ucol])
            ib = plsc.load_gather(ibr, [row, zero, icol])
            x = x + ub + ib
            y = 1.0 / (1.0 + jnp.exp(-x))
            res_v[pl.ds(c * CHUNK + g * L, L)] = y
            return carry

        lax.fori_loop(0, GROUPS, group, 0)
    pltpu.sync_copy(res_v, out.at[pl.ds(base, BPW)])


_sc_kernel = functools.partial(
    pl.kernel,
    out_type=jax.ShapeDtypeStruct((B,), jnp.float32),
    mesh=plsc.VectorSubcoreMesh(core_axis_name="c", subcore_axis_name="s"),
    compiler_params=pltpu.CompilerParams(
        needs_layout_passes=False, use_tc_tiling_on_sc=False,
        skip_device_barrier=True, disable_bounds_checks=True,
        disable_semaphore_checks=True),
    scratch_types=[
        pltpu.VMEM((NCHUNK, CHUNK), jnp.int32),
        pltpu.VMEM((NCHUNK, CHUNK), jnp.int32),
        pltpu.VMEM((NCHUNK, CHUNK), jnp.int32),
        pltpu.VMEM((NCHUNK, CHUNK), jnp.int32),
        pltpu.VMEM((NBUF, CHUNK, D), jnp.float32),
        pltpu.VMEM((NBUF, CHUNK, D), jnp.float32),
        pltpu.VMEM((NBUF, CHUNK, 1, L), jnp.float32),
        pltpu.VMEM((NBUF, CHUNK, 1, L), jnp.float32),
        pltpu.VMEM((L * PITCH,), jnp.float32),
        pltpu.VMEM((BPW,), jnp.float32),
        pltpu.SemaphoreType.DMA((NBUF,)),
    ],
)(_sc_body)


def kernel(user_ids, item_ids, user_emb, item_emb, user_bias, item_bias):
    ub16 = user_bias.reshape(-1, 1, L)
    ib16 = item_bias.reshape(-1, 1, L)
    out = _sc_kernel(user_ids.astype(jnp.int32), item_ids.astype(jnp.int32),
                     user_emb, item_emb, ub16, ib16)
    return out.reshape(B, 1)


# speedup vs baseline: 2.6208x; 1.0082x over previous
import functools

import jax
import jax.numpy as jnp
from jax import lax
from jax.experimental import pallas as pl
from jax.experimental.pallas import tpu as pltpu
from jax.experimental.pallas import tpu_sc as plsc

NC = 2
NS = 16
L = 16
NW = NC * NS

B = 16384
D = 128
BPW = B // NW
CHUNK = 128
NCHUNK = BPW // CHUNK
GROUPS = CHUNK // L
NBUF = 3
PITCH = L + 1


def _sc_body(uids, iids, uemb, iemb, ubias, ibias, out,
             uidx_v, iidx_v, ubrow_v, ibrow_v,
             urows, irows, ubw, ibw, accmat, res_v, sems):
    wid = lax.axis_index("s") * NC + lax.axis_index("c")
    base = wid * BPW
    def stage_ids(c):
        pltpu.sync_copy(uids.at[pl.ds(base + c * CHUNK, CHUNK)], uidx_v.at[c])
        pltpu.sync_copy(iids.at[pl.ds(base + c * CHUNK, CHUNK)], iidx_v.at[c])
        for g in range(GROUPS):
            idu = uidx_v[c, pl.ds(g * L, L)]
            idi = iidx_v[c, pl.ds(g * L, L)]
            ubrow_v[c, pl.ds(g * L, L)] = lax.shift_right_logical(idu, 4)
            ibrow_v[c, pl.ds(g * L, L)] = lax.shift_right_logical(idi, 4)

    def fire(c):
        s = c % NBUF
        return [
            pltpu.async_copy(uemb.at[uidx_v.at[c]], urows.at[s], sems.at[s]),
            pltpu.async_copy(iemb.at[iidx_v.at[c]], irows.at[s], sems.at[s]),
            pltpu.async_copy(ubias.at[ubrow_v.at[c]], ubw.at[s], sems.at[s]),
            pltpu.async_copy(ibias.at[ibrow_v.at[c]], ibw.at[s], sems.at[s]),
        ]

    stage_ids(0)
    handles = {0: fire(0)}
    if NCHUNK > 1:
        stage_ids(1)
        handles[1] = fire(1)
    for c in range(NCHUNK):
        s = c % NBUF
        if c + 2 < NCHUNK:
            stage_ids(c + 2)
            handles[c + 2] = fire(c + 2)
        for h in handles.pop(c):
            h.wait()
        ur = urows.at[s]
        ir = irows.at[s]
        ubr = ubw.at[s]
        ibr = ibw.at[s]

        def group(g, carry, ur=ur, ir=ir, ubr=ubr, ibr=ibr, c=c):
            row0 = g * L
            for r in range(L):
                acc = ur[row0 + r, pl.ds(0, L)] * ir[row0 + r, pl.ds(0, L)]
                for j in range(1, D // L):
                    acc = acc + (ur[row0 + r, pl.ds(j * L, L)]
                                 * ir[row0 + r, pl.ds(j * L, L)])
                accmat[pl.ds(r * PITCH, L)] = acc
            lanes = lax.iota(jnp.int32, L) * PITCH
            x = plsc.load_gather(accmat, [lanes])
            for l in range(1, L):
                x = x + plsc.load_gather(accmat, [lanes + l])
            row = row0 + lax.iota(jnp.int32, L)
            zero = jnp.zeros((L,), jnp.int32)
            ucol = jnp.bitwise_and(uidx_v[c, pl.ds(g * L, L)], 15)
            icol = jnp.bitwise_and(iidx_v[c, pl.ds(g * L, L)], 15)
            ub = plsc.load_gather(ubr, [row, zero, ucol])
            ib = plsc.load_gather(ibr, [row, zero, icol])
            x = x + ub + ib
            y = 1.0 / (1.0 + jnp.exp(-x))
            res_v[pl.ds(c * CHUNK + g * L, L)] = y
            return carry

        lax.fori_loop(0, GROUPS, group, 0)
    pltpu.sync_copy(res_v, out.at[pl.ds(base, BPW)])


_sc_kernel = functools.partial(
    pl.kernel,
    out_type=jax.ShapeDtypeStruct((B,), jnp.float32),
    mesh=plsc.VectorSubcoreMesh(core_axis_name="c", subcore_axis_name="s"),
    compiler_params=pltpu.CompilerParams(
        needs_layout_passes=False, use_tc_tiling_on_sc=False,
        skip_device_barrier=True, disable_bounds_checks=True,
        disable_semaphore_checks=True),
    scratch_types=[
        pltpu.VMEM((NCHUNK, CHUNK), jnp.int32),
        pltpu.VMEM((NCHUNK, CHUNK), jnp.int32),
        pltpu.VMEM((NCHUNK, CHUNK), jnp.int32),
        pltpu.VMEM((NCHUNK, CHUNK), jnp.int32),
        pltpu.VMEM((NBUF, CHUNK, D), jnp.float32),
        pltpu.VMEM((NBUF, CHUNK, D), jnp.float32),
        pltpu.VMEM((NBUF, CHUNK, 1, L), jnp.float32),
        pltpu.VMEM((NBUF, CHUNK, 1, L), jnp.float32),
        pltpu.VMEM((L * PITCH,), jnp.float32),
        pltpu.VMEM((BPW,), jnp.float32),
        pltpu.SemaphoreType.DMA((NBUF,)),
    ],
)(_sc_body)


def kernel(user_ids, item_ids, user_emb, item_emb, user_bias, item_bias):
    ub16 = user_bias.reshape(-1, 1, L)
    ib16 = item_bias.reshape(-1, 1, L)
    out = _sc_kernel(user_ids.astype(jnp.int32), item_ids.astype(jnp.int32),
                     user_emb, item_emb, ub16, ib16)
    return out.reshape(B, 1)
